# baseline (speedup 1.0000x reference)
import functools

import jax
import jax.numpy as jnp
from jax import lax
from jax.experimental import pallas as pl
from jax.experimental.pallas import tpu as pltpu
from jax.experimental.pallas import tpu_sc as plsc

N = 10000
D = 128
E = 320000
NC = 2
NS = 16
NW = NC * NS
CHUNK = 128
CH = 80
NCH = NW * CH
CH0 = 128
CH1 = 32
P = 32
EP = NCH * CHUNK
NPAD = 10240
STRIPE = NPAD // NS
TRASH = 10016

_mesh = plsc.VectorSubcoreMesh(core_axis_name="c", subcore_axis_name="s")


@functools.partial(
    pl.kernel,
    out_type=jax.ShapeDtypeStruct((NC * NPAD,), jnp.float32),
    mesh=_mesh,
    scratch_types=[
        pltpu.VMEM((CH, CHUNK), jnp.int32),
        pltpu.VMEM((CHUNK,), jnp.float32),
        pltpu.VMEM((STRIPE,), jnp.float32),
        pltpu.VMEM_SHARED((NPAD,), jnp.float32),
    ],
)
def _deg_kernel(dst_hbm, zeros_hbm, out_hbm, dstv, ones_v, zer_v, deg_sh):
    cid = lax.axis_index("c")
    sid = lax.axis_index("s")
    wid = sid * NC + cid
    for i in range(CHUNK // 16):
        ones_v[pl.ds(i * 16, 16)] = jnp.ones((16,), jnp.float32)
    pltpu.sync_copy(zeros_hbm, zer_v)
    pltpu.sync_copy(zer_v, deg_sh.at[pl.ds(sid * STRIPE, STRIPE)])
    plsc.subcore_barrier()
    pltpu.sync_copy(dst_hbm.at[wid], dstv)

    def body(j, c):
        pltpu.sync_copy(ones_v, deg_sh.at[dstv.at[j]], add=True)
        return c

    lax.fori_loop(0, CH, body, 0)
    plsc.subcore_barrier()
    pltpu.sync_copy(deg_sh.at[pl.ds(sid * STRIPE, STRIPE)], zer_v)
    pltpu.sync_copy(zer_v, out_hbm.at[pl.ds(cid * NPAD + sid * STRIPE, STRIPE)])


@functools.partial(
    pl.kernel,
    out_type=jax.ShapeDtypeStruct((NC, NPAD, D), jnp.float32),
    mesh=_mesh,
    scratch_types=[
        pltpu.VMEM((2 * P, CHUNK // 2), jnp.int32),
        pltpu.VMEM((P, CHUNK), jnp.int32),
        pltpu.VMEM((CHUNK, D), jnp.float32),
        pltpu.VMEM((CHUNK, D), jnp.float32),
        pltpu.VMEM_SHARED((NPAD, D), jnp.float32),
        pltpu.SemaphoreType.DMA,
        pltpu.SemaphoreType.DMA,
        pltpu.SemaphoreType.DMA,
        pltpu.SemaphoreType.DMA,
    ],
)
def _msg_kernel(y_hbm, src2_hbm, src_hbm, dst_hbm, zeros_hbm, out_hbm,
                srcv, dstv, rows0, rows1, acc_sh,
                sem0a, sem0b, sem1a, sem1b):
    cid = lax.axis_index("c")
    sid = lax.axis_index("s")
    pltpu.sync_copy(zeros_hbm, rows0)

    def zbody(t, c):
        pltpu.sync_copy(rows0, acc_sh.at[pl.ds(sid * STRIPE + t * CHUNK, CHUNK)])
        return c

    lax.fori_loop(0, STRIPE // CHUNK, zbody, 0)
    plsc.subcore_barrier()

    base = jnp.where(cid == 0, sid * CH0, NS * CH0 + sid * CH1)
    npieces = jnp.where(cid == 0, CH0 // P, CH1 // P)

    Q = CHUNK // 2

    def gat(j, buf, sa, sb):
        for q in range(2):
            pltpu.async_copy(y_hbm.at[srcv.at[2 * j + q]],
                             buf.at[pl.ds(q * Q, Q)], sa if q < 1 else sb)

    def wat(j, buf, sa, sb):
        for q in range(2):
            pltpu.make_async_copy(y_hbm.at[srcv.at[2 * j + q]],
                                  buf.at[pl.ds(q * Q, Q)],
                                  sa if q < 1 else sb).wait()

    def piece(p, c):
        pltpu.sync_copy(src2_hbm.at[pl.ds(2 * (base + p * P), 2 * P)], srcv)
        pltpu.sync_copy(dst_hbm.at[pl.ds(base + p * P, P)], dstv)
        gat(0, rows0, sem0a, sem0b)

        def body(j0, c2):
            j = 2 * j0
            gat(j + 1, rows1, sem1a, sem1b)
            wat(j, rows0, sem0a, sem0b)
            pltpu.sync_copy(rows0, acc_sh.at[dstv.at[j]], add=True)

            @pl.when(j0 < P // 2 - 1)
            def _():
                gat(j + 2, rows0, sem0a, sem0b)

            wat(j + 1, rows1, sem1a, sem1b)
            pltpu.sync_copy(rows1, acc_sh.at[dstv.at[j + 1]], add=True)
            return c2

        lax.fori_loop(0, P // 2, body, 0)
        return c

    lax.fori_loop(0, npieces, piece, 0)
    plsc.subcore_barrier()

    def wbody(t, c):
        r = sid * STRIPE + t * CHUNK
        pltpu.sync_copy(acc_sh.at[pl.ds(r, CHUNK)], rows0)
        pltpu.sync_copy(rows0, out_hbm.at[cid, pl.ds(r, CHUNK)])
        return c

    lax.fori_loop(0, STRIPE // CHUNK, wbody, 0)


BM = 400


def _dinv(dgt_ref):
    return lax.rsqrt(dgt_ref[:, 0:1] + dgt_ref[:, 1:2] + 1.0)


def _elu(v):
    return jnp.where(v > 0.0, v, jnp.exp(v) - 1.0)


def _tca_body(x_ref, w_ref, dgt_ref, y_ref):
    y_ref[...] = jnp.dot(x_ref[...], w_ref[...],
                         preferred_element_type=jnp.float32) * _dinv(dgt_ref)


def _tcb_body(mp_ref, y1_ref, dgt_ref, b_ref, w_ref, y2_ref):
    dv = _dinv(dgt_ref)
    h = _elu(dv * (mp_ref[0] + mp_ref[1] + y1_ref[...]) + b_ref[...])
    y2_ref[...] = jnp.dot(h, w_ref[...],
                          preferred_element_type=jnp.float32) * dv


def _tcc_body(mq_ref, y2_ref, dgt_ref, b_ref, o_ref):
    dv = _dinv(dgt_ref)
    o_ref[...] = _elu(dv * (mq_ref[0] + mq_ref[1] + y2_ref[...]) + b_ref[...])


_row = pl.BlockSpec((BM, D), lambda i: (i, 0))
_full = pl.BlockSpec((D, D), lambda i: (0, 0))
_dgt = pl.BlockSpec((BM, 2), lambda i: (i, 0))
_bias = pl.BlockSpec((1, D), lambda i: (0, 0))
_part = pl.BlockSpec((NC, BM, D), lambda i: (0, i, 0))
_osd = jax.ShapeDtypeStruct((N, D), jnp.float32)

_tca = pl.pallas_call(_tca_body, grid=(N // BM,), out_shape=_osd,
                      in_specs=[_row, _full, _dgt], out_specs=_row)
_tcb = pl.pallas_call(_tcb_body, grid=(N // BM,), out_shape=_osd,
                      in_specs=[_part, _row, _dgt, _bias, _full], out_specs=_row)
_tcc = pl.pallas_call(_tcc_body, grid=(N // BM,), out_shape=_osd,
                      in_specs=[_part, _row, _dgt, _bias], out_specs=_row)


def kernel(x, edge_index, W1, b1, W2, b2):
    src = edge_index[0].astype(jnp.int32)
    dst = edge_index[1].astype(jnp.int32)
    src_w = jnp.concatenate(
        [src, jnp.zeros((EP - E,), jnp.int32)]).reshape(NW, CH, CHUNK)
    dst_w = jnp.concatenate(
        [dst, jnp.full((EP - E,), TRASH, jnp.int32)]).reshape(NW, CH, CHUNK)
    src_f = src_w.reshape(NCH, CHUNK)
    dst_f = dst_w.reshape(NCH, CHUNK)
    zeros1 = jnp.zeros((STRIPE,), jnp.float32)
    zeros2 = jnp.zeros((CHUNK, D), jnp.float32)
    b1r = b1.reshape(1, D)
    b2r = b2.reshape(1, D)

    degp = _deg_kernel(dst_w, zeros1)
    degp_t = degp.reshape(NC, NPAD).T

    src_h2 = src_f.reshape(NCH * 2, CHUNK // 2)

    y1 = _tca(x, W1, degp_t)
    mp = _msg_kernel(y1, src_h2, src_f, dst_f, zeros2)
    y2 = _tcb(mp, y1, degp_t, b1r, W2)
    mq = _msg_kernel(y2, src_h2, src_f, dst_f, zeros2)
    return _tcc(mq, y2, degp_t, b2r)


# speedup vs baseline: 1.0811x; 1.0811x over previous
import functools

import jax
import jax.numpy as jnp
from jax import lax
from jax.experimental import pallas as pl
from jax.experimental.pallas import tpu as pltpu
from jax.experimental.pallas import tpu_sc as plsc

N = 10000
D = 128
E = 320000
NC = 2
NS = 16
NW = NC * NS
CHUNK = 128
CH = 80
NCH = NW * CH
CH0 = 144
CH1 = 16
P = 16
EP = NCH * CHUNK
NPAD = 10240
STRIPE = NPAD // NS
TRASH = 10016

_mesh = plsc.VectorSubcoreMesh(core_axis_name="c", subcore_axis_name="s")


@functools.partial(
    pl.kernel,
    out_type=jax.ShapeDtypeStruct((NC * NPAD,), jnp.float32),
    mesh=_mesh,
    scratch_types=[
        pltpu.VMEM((CH, CHUNK), jnp.int32),
        pltpu.VMEM((CHUNK,), jnp.float32),
        pltpu.VMEM((STRIPE,), jnp.float32),
        pltpu.VMEM_SHARED((NPAD,), jnp.float32),
    ],
)
def _deg_kernel(dst_hbm, zeros_hbm, out_hbm, dstv, ones_v, zer_v, deg_sh):
    cid = lax.axis_index("c")
    sid = lax.axis_index("s")
    wid = sid * NC + cid
    for i in range(CHUNK // 16):
        ones_v[pl.ds(i * 16, 16)] = jnp.ones((16,), jnp.float32)
    pltpu.sync_copy(zeros_hbm, zer_v)
    pltpu.sync_copy(zer_v, deg_sh.at[pl.ds(sid * STRIPE, STRIPE)])
    plsc.subcore_barrier()
    pltpu.sync_copy(dst_hbm.at[wid], dstv)

    def body(j, c):
        pltpu.sync_copy(ones_v, deg_sh.at[dstv.at[j]], add=True)
        return c

    lax.fori_loop(0, CH, body, 0)
    plsc.subcore_barrier()
    pltpu.sync_copy(deg_sh.at[pl.ds(sid * STRIPE, STRIPE)], zer_v)
    pltpu.sync_copy(zer_v, out_hbm.at[pl.ds(cid * NPAD + sid * STRIPE, STRIPE)])


@functools.partial(
    pl.kernel,
    out_type=jax.ShapeDtypeStruct((NC, NPAD, D), jnp.float32),
    mesh=_mesh,
    scratch_types=[
        pltpu.VMEM((2 * P, CHUNK // 2), jnp.int32),
        pltpu.VMEM((P, CHUNK), jnp.int32),
        pltpu.VMEM((CHUNK, D), jnp.float32),
        pltpu.VMEM((CHUNK, D), jnp.float32),
        pltpu.VMEM_SHARED((NPAD, D), jnp.float32),
        pltpu.SemaphoreType.DMA,
        pltpu.SemaphoreType.DMA,
        pltpu.SemaphoreType.DMA,
        pltpu.SemaphoreType.DMA,
    ],
)
def _msg_kernel(y_hbm, src2_hbm, src_hbm, dst_hbm, zeros_hbm, out_hbm,
                srcv, dstv, rows0, rows1, acc_sh,
                sem0a, sem0b, sem1a, sem1b):
    cid = lax.axis_index("c")
    sid = lax.axis_index("s")
    pltpu.sync_copy(zeros_hbm, rows0)

    def zbody(t, c):
        pltpu.sync_copy(rows0, acc_sh.at[pl.ds(sid * STRIPE + t * CHUNK, CHUNK)])
        return c

    lax.fori_loop(0, STRIPE // CHUNK, zbody, 0)
    plsc.subcore_barrier()

    base = jnp.where(cid == 0, sid * CH0, NS * CH0 + sid * CH1)
    npieces = jnp.where(cid == 0, CH0 // P, CH1 // P)

    Q = CHUNK // 2

    def gat(j, buf, sa, sb):
        for q in range(2):
            pltpu.async_copy(y_hbm.at[srcv.at[2 * j + q]],
                             buf.at[pl.ds(q * Q, Q)], sa if q < 1 else sb)

    def wat(j, buf, sa, sb):
        for q in range(2):
            pltpu.make_async_copy(y_hbm.at[srcv.at[2 * j + q]],
                                  buf.at[pl.ds(q * Q, Q)],
                                  sa if q < 1 else sb).wait()

    def piece(p, c):
        pltpu.sync_copy(src2_hbm.at[pl.ds(2 * (base + p * P), 2 * P)], srcv)
        pltpu.sync_copy(dst_hbm.at[pl.ds(base + p * P, P)], dstv)
        gat(0, rows0, sem0a, sem0b)

        def body(j0, c2):
            j = 2 * j0
            gat(j + 1, rows1, sem1a, sem1b)
            wat(j, rows0, sem0a, sem0b)
            pltpu.sync_copy(rows0, acc_sh.at[dstv.at[j]], add=True)

            @pl.when(j0 < P // 2 - 1)
            def _():
                gat(j + 2, rows0, sem0a, sem0b)

            wat(j + 1, rows1, sem1a, sem1b)
            pltpu.sync_copy(rows1, acc_sh.at[dstv.at[j + 1]], add=True)
            return c2

        lax.fori_loop(0, P // 2, body, 0)
        return c

    lax.fori_loop(0, npieces, piece, 0)
    plsc.subcore_barrier()

    def wbody(t, c):
        r = sid * STRIPE + t * CHUNK
        pltpu.sync_copy(acc_sh.at[pl.ds(r, CHUNK)], rows0)
        pltpu.sync_copy(rows0, out_hbm.at[cid, pl.ds(r, CHUNK)])
        return c

    lax.fori_loop(0, STRIPE // CHUNK, wbody, 0)


BM = 400


def _dinv(dgt_ref):
    return lax.rsqrt(dgt_ref[:, 0:1] + dgt_ref[:, 1:2] + 1.0)


def _elu(v):
    return jnp.where(v > 0.0, v, jnp.exp(v) - 1.0)


def _tca_body(x_ref, w_ref, dgt_ref, y_ref):
    y_ref[...] = jnp.dot(x_ref[...], w_ref[...],
                         preferred_element_type=jnp.float32) * _dinv(dgt_ref)


def _tcb_body(mp_ref, y1_ref, dgt_ref, b_ref, w_ref, y2_ref):
    dv = _dinv(dgt_ref)
    h = _elu(dv * (mp_ref[0] + mp_ref[1] + y1_ref[...]) + b_ref[...])
    y2_ref[...] = jnp.dot(h, w_ref[...],
                          preferred_element_type=jnp.float32) * dv


def _tcc_body(mq_ref, y2_ref, dgt_ref, b_ref, o_ref):
    dv = _dinv(dgt_ref)
    o_ref[...] = _elu(dv * (mq_ref[0] + mq_ref[1] + y2_ref[...]) + b_ref[...])


_row = pl.BlockSpec((BM, D), lambda i: (i, 0))
_full = pl.BlockSpec((D, D), lambda i: (0, 0))
_dgt = pl.BlockSpec((BM, 2), lambda i: (i, 0))
_bias = pl.BlockSpec((1, D), lambda i: (0, 0))
_part = pl.BlockSpec((NC, BM, D), lambda i: (0, i, 0))
_osd = jax.ShapeDtypeStruct((N, D), jnp.float32)

_tca = pl.pallas_call(_tca_body, grid=(N // BM,), out_shape=_osd,
                      in_specs=[_row, _full, _dgt], out_specs=_row)
_tcb = pl.pallas_call(_tcb_body, grid=(N // BM,), out_shape=_osd,
                      in_specs=[_part, _row, _dgt, _bias, _full], out_specs=_row)
_tcc = pl.pallas_call(_tcc_body, grid=(N // BM,), out_shape=_osd,
                      in_specs=[_part, _row, _dgt, _bias], out_specs=_row)


def kernel(x, edge_index, W1, b1, W2, b2):
    src = edge_index[0].astype(jnp.int32)
    dst = edge_index[1].astype(jnp.int32)
    src_w = jnp.concatenate(
        [src, jnp.zeros((EP - E,), jnp.int32)]).reshape(NW, CH, CHUNK)
    dst_w = jnp.concatenate(
        [dst, jnp.full((EP - E,), TRASH, jnp.int32)]).reshape(NW, CH, CHUNK)
    src_f = src_w.reshape(NCH, CHUNK)
    dst_f = dst_w.reshape(NCH, CHUNK)
    zeros1 = jnp.zeros((STRIPE,), jnp.float32)
    zeros2 = jnp.zeros((CHUNK, D), jnp.float32)
    b1r = b1.reshape(1, D)
    b2r = b2.reshape(1, D)

    degp = _deg_kernel(dst_w, zeros1)
    degp_t = degp.reshape(NC, NPAD).T

    src_h2 = src_f.reshape(NCH * 2, CHUNK // 2)

    y1 = _tca(x, W1, degp_t)
    mp = _msg_kernel(y1, src_h2, src_f, dst_f, zeros2)
    y2 = _tcb(mp, y1, degp_t, b1r, W2)
    mq = _msg_kernel(y2, src_h2, src_f, dst_f, zeros2)
    return _tcc(mq, y2, degp_t, b2r)
